# baseline (speedup 1.0000x reference)
import dataclasses
import functools

import jax
import jax.numpy as jnp
from jax import lax
from jax.experimental import pallas as pl
from jax.experimental.pallas import tpu as pltpu
from jax.experimental.pallas import tpu_sc as plsc

B, C, H, W = 2, 96, 384, 384
P = B * H * W
NREC = 2 * P
NSUB = 16
NCORE = 2
REC_PER_SUB = NREC // NSUB
SCAN = 2048
NSCAN = REC_PER_SUB // SCAN
CHUNK = 32 * W
NPASS = (P // NCORE) // CHUNK
ROWS_PER_SUB = CHUNK // NSUB
TILE = 64

def _iota16():
    return lax.broadcasted_iota(jnp.int32, (16,), 0)


def _records_body(fx_ref, fy_ref, d_ref, wl_ref, wr_ref):
    i = pl.program_id(0)
    fx = fx_ref[...]
    fy = fy_ref[...]
    r_i = lax.broadcasted_iota(jnp.int32, (8, W), 0) + i * 8
    b = jnp.where(r_i >= H, 1, 0)
    y = r_i - b * H
    x = lax.broadcasted_iota(jnp.int32, (8, W), 1)
    fxc = jnp.clip(fx, -2.0 * W, 2.0 * W)
    fyc = jnp.clip(fy, -2.0 * W, 2.0 * W)
    gx = x.astype(jnp.float32) + fxc
    gy = y.astype(jnp.float32) + fyc
    x_f = jnp.floor(gx)
    y_f = jnp.floor(gy)
    x_c = x_f + 1.0
    y_c = y_f + 1.0
    nw = (x_c - gx) * (y_c - gy)
    ne = (gx - x_f) * (y_c - gy)
    sw = (x_c - gx) * (gy - y_f)
    se = (gx - x_f) * (gy - y_f)
    xfi = x_f.astype(jnp.int32)
    yfi = y_f.astype(jnp.int32)
    base = b * (H * W)
    vx = (xfi >= -1) & (xfi <= W - 1)
    dx = jnp.where(xfi >= 0, xfi, 0)
    for plane, (yp, wlr, wrr) in enumerate(((yfi, nw, ne), (yfi + 1, sw, se))):
        vy = (yp >= 0) & (yp < H)
        valid = vy & vx
        wl = jnp.where(xfi >= 0, wlr, wrr)
        wr = jnp.where((xfi >= 0) & (xfi <= W - 2), wrr, 0.0)
        d = jnp.where(valid, base + yp * W + dx, -1)
        d_ref[plane] = d
        wl_ref[plane] = jnp.where(valid, wl, 0.0)
        wr_ref[plane] = jnp.where(valid, wr, 0.0)


def _build_records(fx, fy):
    grid = (B * H // 8,)
    return pl.pallas_call(
        _records_body,
        grid=grid,
        in_specs=[pl.BlockSpec((8, W), lambda i: (i, 0)),
                  pl.BlockSpec((8, W), lambda i: (i, 0))],
        out_specs=[pl.BlockSpec((2, 8, W), lambda i: (0, i, 0)),
                   pl.BlockSpec((2, 8, W), lambda i: (0, i, 0)),
                   pl.BlockSpec((2, 8, W), lambda i: (0, i, 0))],
        out_shape=[jax.ShapeDtypeStruct((2, B * H, W), jnp.int32),
                   jax.ShapeDtypeStruct((2, B * H, W), jnp.float32),
                   jax.ShapeDtypeStruct((2, B * H, W), jnp.float32)],
    )(fx, fy)


def _sc_body(im0p, d2, wl2, wr2, outp,
             acc, scan_d, scan_wl, scan_wr,
             ld, ld1, lp, lwl, lwr,
             ibp, ibd, ibd1, stag_in, stag_l, stag_r, zeros):
    c = lax.axis_index("c")
    s = lax.axis_index("s")
    p_base = (s % 8) * REC_PER_SUB
    rec_base = s * REC_PER_SUB

    z16f = jnp.zeros((16,), jnp.float32)
    z16i = jnp.zeros((16,), jnp.int32)

    @pl.loop(0, 128)
    def _zfill(r):
        for c6 in range(6):
            zeros[r, pl.ds(c6 * 16, 16)] = z16f

    @pl.loop(0, NPASS)
    def _pass(pi):
        chunk_lo = c * (P // NCORE) + pi * CHUNK

        @pl.loop(0, ROWS_PER_SUB // 128)
        def _zero(k):
            pltpu.sync_copy(zeros, acc.at[pl.ds(s * ROWS_PER_SUB + k * 128, 128)])

        @pl.when(s == NSUB - 1)
        def _zero_guard():
            pltpu.sync_copy(zeros.at[pl.ds(0, 1)], acc.at[pl.ds(CHUNK, 1)])

        plsc.subcore_barrier()

        @pl.loop(0, NSCAN)
        def _scan(si):
            off = rec_base + si * SCAN
            pltpu.sync_copy(d2.at[pl.ds(off, SCAN)], scan_d)
            pltpu.sync_copy(wl2.at[pl.ds(off, SCAN)], scan_wl)
            pltpu.sync_copy(wr2.at[pl.ds(off, SCAN)], scan_wr)
            p0 = p_base + si * SCAN

            def vec_body(v, cursor):
                d = scan_d[pl.ds(v * 16, 16)]
                m = (d >= chunk_lo) & (d < chunk_lo + CHUNK)
                dl = d - chunk_lo
                plsc.store_compressed(ld.at[pl.ds(cursor, 16)], dl, mask=m)
                plsc.store_compressed(ld1.at[pl.ds(cursor, 16)], dl + 1, mask=m)
                pv = p0 + v * 16 + _iota16()
                plsc.store_compressed(lp.at[pl.ds(cursor, 16)], pv, mask=m)
                plsc.store_compressed(lwl.at[pl.ds(cursor, 16)],
                                      scan_wl[pl.ds(v * 16, 16)], mask=m)
                plsc.store_compressed(lwr.at[pl.ds(cursor, 16)],
                                      scan_wr[pl.ds(v * 16, 16)], mask=m)
                return cursor + jnp.sum(m.astype(jnp.int32))

            cursor = lax.fori_loop(0, SCAN // 16, vec_body, jnp.int32(0))

            @pl.loop(0, TILE // 16)
            def _pad(j):
                ld[pl.ds(cursor + j * 16, 16)] = z16i
                ld1[pl.ds(cursor + j * 16, 16)] = z16i
                lp[pl.ds(cursor + j * 16, 16)] = z16i
                lwl[pl.ds(cursor + j * 16, 16)] = z16f
                lwr[pl.ds(cursor + j * 16, 16)] = z16f

            ntiles = (cursor + TILE - 1) // TILE

            def tile_body(t, carry):
                tb = t * TILE

                @pl.loop(0, TILE // 16)
                def _cp(k):
                    ibp[pl.ds(k * 16, 16)] = lp[pl.ds(tb + k * 16, 16)]
                    ibd[pl.ds(k * 16, 16)] = ld[pl.ds(tb + k * 16, 16)]
                    ibd1[pl.ds(k * 16, 16)] = ld1[pl.ds(tb + k * 16, 16)]

                pltpu.sync_copy(im0p.at[ibp], stag_in)

                @pl.loop(0, TILE)
                def _scale(r):
                    idxv = jnp.full((16,), tb + r, jnp.int32)
                    wlv = plsc.load_gather(lwl, [idxv])
                    wrv = plsc.load_gather(lwr, [idxv])
                    for c6 in range(6):
                        v = stag_in[r, pl.ds(c6 * 16, 16)]
                        stag_l[r, pl.ds(c6 * 16, 16)] = v * wlv
                        stag_r[r, pl.ds(c6 * 16, 16)] = v * wrv

                pltpu.sync_copy(stag_l, acc.at[ibd], add=True)
                pltpu.sync_copy(stag_r, acc.at[ibd1], add=True)
                return carry

            lax.fori_loop(0, ntiles, tile_body, jnp.int32(0))

        plsc.subcore_barrier()
        pltpu.sync_copy(acc.at[pl.ds(s * ROWS_PER_SUB, ROWS_PER_SUB)],
                        outp.at[pl.ds(chunk_lo + s * ROWS_PER_SUB, ROWS_PER_SUB)])
        plsc.subcore_barrier()


_SC_SCATTER = None


def _get_sc_scatter():
    global _SC_SCATTER
    if _SC_SCATTER is None:
        _SC_SCATTER = _make_sc_scatter()
    return _SC_SCATTER


def _make_sc_scatter():
    cp = pltpu.CompilerParams()
    if "needs_layout_passes" in pltpu.CompilerParams.__dataclass_fields__:
        cp = dataclasses.replace(cp, needs_layout_passes=False)
    if "use_tc_tiling_on_sc" in pltpu.CompilerParams.__dataclass_fields__:
        cp = dataclasses.replace(cp, use_tc_tiling_on_sc=False)
    return pl.kernel(
        _sc_body,
        out_type=jax.ShapeDtypeStruct((P, C), jnp.float32),
        mesh=plsc.VectorSubcoreMesh(core_axis_name="c", subcore_axis_name="s"),
        compiler_params=cp,
        scratch_types=[
        pltpu.VMEM_SHARED((CHUNK + 1, C), jnp.float32),
        pltpu.VMEM((SCAN,), jnp.int32),
        pltpu.VMEM((SCAN,), jnp.float32),
        pltpu.VMEM((SCAN,), jnp.float32),
        pltpu.VMEM((SCAN + TILE,), jnp.int32),
        pltpu.VMEM((SCAN + TILE,), jnp.int32),
        pltpu.VMEM((SCAN + TILE,), jnp.int32),
        pltpu.VMEM((SCAN + TILE,), jnp.float32),
        pltpu.VMEM((SCAN + TILE,), jnp.float32),
        pltpu.VMEM((TILE,), jnp.int32),
        pltpu.VMEM((TILE,), jnp.int32),
        pltpu.VMEM((TILE,), jnp.int32),
        pltpu.VMEM((TILE, C), jnp.float32),
        pltpu.VMEM((TILE, C), jnp.float32),
        pltpu.VMEM((TILE, C), jnp.float32),
            pltpu.VMEM((128, C), jnp.float32),
        ],
    )


@jax.jit
def kernel(im0, flow):
    im0p = jnp.transpose(im0, (0, 2, 3, 1)).reshape(P, C)
    fx = flow[..., 0].reshape(B * H, W)
    fy = flow[..., 1].reshape(B * H, W)
    d2, wl2, wr2 = _build_records(fx, fy)
    outp = _get_sc_scatter()(im0p, d2.reshape(NREC), wl2.reshape(NREC),
                             wr2.reshape(NREC))
    return outp.reshape(B, H, W, C).transpose(0, 3, 1, 2)


# speedup vs baseline: 1.0620x; 1.0620x over previous
import dataclasses

import jax
import jax.numpy as jnp
from jax import lax
from jax.experimental import pallas as pl
from jax.experimental.pallas import tpu as pltpu
from jax.experimental.pallas import tpu_sc as plsc

B, C, H, W = 2, 96, 384, 384
P = B * H * W
NREC = 2 * P
NSUB = 16
NCORE = 2
REC_PER_SUB = NREC // NSUB
SCAN = 2048
NSCAN = REC_PER_SUB // SCAN
CHUNK = 32 * W
NPASS = (P // NCORE) // CHUNK
ROWS_PER_SUB = CHUNK // NSUB
TILE = 64


def _iota16():
    return lax.broadcasted_iota(jnp.int32, (16,), 0)


def _records_body(fx_ref, fy_ref, d_ref, wl_ref, wr_ref):
    i = pl.program_id(0)
    fx = fx_ref[...]
    fy = fy_ref[...]
    r_i = lax.broadcasted_iota(jnp.int32, (8, W), 0) + i * 8
    b = jnp.where(r_i >= H, 1, 0)
    y = r_i - b * H
    x = lax.broadcasted_iota(jnp.int32, (8, W), 1)
    fxc = jnp.clip(fx, -2.0 * W, 2.0 * W)
    fyc = jnp.clip(fy, -2.0 * W, 2.0 * W)
    gx = x.astype(jnp.float32) + fxc
    gy = y.astype(jnp.float32) + fyc
    x_f = jnp.floor(gx)
    y_f = jnp.floor(gy)
    x_c = x_f + 1.0
    y_c = y_f + 1.0
    nw = (x_c - gx) * (y_c - gy)
    ne = (gx - x_f) * (y_c - gy)
    sw = (x_c - gx) * (gy - y_f)
    se = (gx - x_f) * (gy - y_f)
    xfi = x_f.astype(jnp.int32)
    yfi = y_f.astype(jnp.int32)
    base = b * (H * W)
    vx = (xfi >= -1) & (xfi <= W - 1)
    dx = jnp.where(xfi >= 0, xfi, 0)
    for plane, (yp, wlr, wrr) in enumerate(((yfi, nw, ne), (yfi + 1, sw, se))):
        vy = (yp >= 0) & (yp < H)
        valid = vy & vx
        wl = jnp.where(xfi >= 0, wlr, wrr)
        wr = jnp.where((xfi >= 0) & (xfi <= W - 2), wrr, 0.0)
        d = jnp.where(valid, base + yp * W + dx, -1)
        d_ref[plane] = d
        wl_ref[plane] = jnp.where(valid, wl, 0.0)
        wr_ref[plane] = jnp.where(valid, wr, 0.0)


def _build_records(fx, fy):
    return pl.pallas_call(
        _records_body,
        grid=(B * H // 8,),
        in_specs=[pl.BlockSpec((8, W), lambda i: (i, 0)),
                  pl.BlockSpec((8, W), lambda i: (i, 0))],
        out_specs=[pl.BlockSpec((2, 8, W), lambda i: (0, i, 0)),
                   pl.BlockSpec((2, 8, W), lambda i: (0, i, 0)),
                   pl.BlockSpec((2, 8, W), lambda i: (0, i, 0))],
        out_shape=[jax.ShapeDtypeStruct((2, B * H, W), jnp.int32),
                   jax.ShapeDtypeStruct((2, B * H, W), jnp.float32),
                   jax.ShapeDtypeStruct((2, B * H, W), jnp.float32)],
    )(fx, fy)


def _sc_body(im0p, d2, wl2, wr2, outp,
             acc, scan_d0, scan_wl0, scan_wr0, scan_d1, scan_wl1, scan_wr1,
             ld, ld1, lp, lwl, lwr,
             ibp, ibd, ibd1, stag_in, stag_l, stag_r, zeros,
             zsem, da0, wa0, ra0, da1, wa1, ra1):
    c = lax.axis_index("c")
    s = lax.axis_index("s")
    p_base = (s % 8) * REC_PER_SUB
    rec_base = s * REC_PER_SUB

    z16f = jnp.zeros((16,), jnp.float32)
    z16i = jnp.zeros((16,), jnp.int32)

    @pl.loop(0, 128)
    def _zfill(r):
        for c6 in range(6):
            zeros[r, pl.ds(c6 * 16, 16)] = z16f

    scan_bufs = ((scan_d0, scan_wl0, scan_wr0, da0, wa0, ra0),
                 (scan_d1, scan_wl1, scan_wr1, da1, wa1, ra1))

    def _fetch(si, bufs):
        sd, swl, swr, dsm, wsm, rsm = bufs
        off = rec_base + jnp.minimum(si, NSCAN - 1) * SCAN
        pltpu.async_copy(d2.at[pl.ds(off, SCAN)], sd, dsm)
        pltpu.async_copy(wl2.at[pl.ds(off, SCAN)], swl, wsm)
        pltpu.async_copy(wr2.at[pl.ds(off, SCAN)], swr, rsm)

    def _wait(bufs):
        sd, swl, swr, dsm, wsm, rsm = bufs
        pltpu.make_async_copy(d2.at[pl.ds(0, SCAN)], sd, dsm).wait()
        pltpu.make_async_copy(wl2.at[pl.ds(0, SCAN)], swl, wsm).wait()
        pltpu.make_async_copy(wr2.at[pl.ds(0, SCAN)], swr, rsm).wait()

    @pl.loop(0, NPASS)
    def _pass(pi):
        chunk_lo = c * (P // NCORE) + pi * CHUNK

        zcs = []
        for k in range(ROWS_PER_SUB // 128):
            zcs.append(pltpu.async_copy(
                zeros, acc.at[pl.ds(s * ROWS_PER_SUB + k * 128, 128)], zsem))
        for zc in zcs:
            zc.wait()

        @pl.when(s == NSUB - 1)
        def _zero_guard():
            pltpu.sync_copy(zeros.at[pl.ds(0, 1)], acc.at[pl.ds(CHUNK, 1)])

        plsc.subcore_barrier()

        _fetch(jnp.int32(0), scan_bufs[0])
        for si in range(NSCAN):
            cur_bufs = scan_bufs[si % 2]
            _fetch(jnp.int32(si + 1), scan_bufs[(si + 1) % 2])
            _wait(cur_bufs)
            sd, swl, swr = cur_bufs[0], cur_bufs[1], cur_bufs[2]
            p0 = p_base + si * SCAN

            def vec_body(v, cursor, sd=sd, swl=swl, swr=swr, p0=p0):
                d = sd[pl.ds(v * 16, 16)]
                m = (d >= chunk_lo) & (d < chunk_lo + CHUNK)
                dl = d - chunk_lo
                plsc.store_compressed(ld.at[pl.ds(cursor, 16)], dl, mask=m)
                plsc.store_compressed(ld1.at[pl.ds(cursor, 16)], dl + 1,
                                      mask=m)
                pv = p0 + v * 16 + _iota16()
                plsc.store_compressed(lp.at[pl.ds(cursor, 16)], pv, mask=m)
                plsc.store_compressed(lwl.at[pl.ds(cursor, 16)],
                                      swl[pl.ds(v * 16, 16)], mask=m)
                plsc.store_compressed(lwr.at[pl.ds(cursor, 16)],
                                      swr[pl.ds(v * 16, 16)], mask=m)
                return cursor + jnp.sum(m.astype(jnp.int32))

            cursor = lax.fori_loop(0, SCAN // 16, vec_body, jnp.int32(0))

            @pl.loop(0, TILE // 16)
            def _pad(j):
                ld[pl.ds(cursor + j * 16, 16)] = z16i
                ld1[pl.ds(cursor + j * 16, 16)] = z16i
                lp[pl.ds(cursor + j * 16, 16)] = z16i
                lwl[pl.ds(cursor + j * 16, 16)] = z16f
                lwr[pl.ds(cursor + j * 16, 16)] = z16f

            ntiles = (cursor + TILE - 1) // TILE

            def tile_body(t, carry):
                tb = t * TILE

                @pl.loop(0, TILE // 16)
                def _cp(k):
                    ibp[pl.ds(k * 16, 16)] = lp[pl.ds(tb + k * 16, 16)]
                    ibd[pl.ds(k * 16, 16)] = ld[pl.ds(tb + k * 16, 16)]
                    ibd1[pl.ds(k * 16, 16)] = ld1[pl.ds(tb + k * 16, 16)]

                pltpu.sync_copy(im0p.at[ibp], stag_in)

                @pl.loop(0, TILE)
                def _scale(r):
                    idxv = jnp.full((16,), tb + r, jnp.int32)
                    wlv = plsc.load_gather(lwl, [idxv])
                    wrv = plsc.load_gather(lwr, [idxv])
                    for c6 in range(6):
                        v = stag_in[r, pl.ds(c6 * 16, 16)]
                        stag_l[r, pl.ds(c6 * 16, 16)] = v * wlv
                        stag_r[r, pl.ds(c6 * 16, 16)] = v * wrv

                pltpu.sync_copy(stag_l, acc.at[ibd], add=True)
                pltpu.sync_copy(stag_r, acc.at[ibd1], add=True)
                return carry

            lax.fori_loop(0, ntiles, tile_body, jnp.int32(0))

        _wait(scan_bufs[NSCAN % 2])

        plsc.subcore_barrier()
        pltpu.sync_copy(acc.at[pl.ds(s * ROWS_PER_SUB, ROWS_PER_SUB)],
                        outp.at[pl.ds(chunk_lo + s * ROWS_PER_SUB,
                                      ROWS_PER_SUB)])
        plsc.subcore_barrier()


_SC_SCATTER = None


def _get_sc_scatter():
    global _SC_SCATTER
    if _SC_SCATTER is None:
        _SC_SCATTER = _make_sc_scatter()
    return _SC_SCATTER


def _make_sc_scatter():
    cp = pltpu.CompilerParams()
    if "needs_layout_passes" in pltpu.CompilerParams.__dataclass_fields__:
        cp = dataclasses.replace(cp, needs_layout_passes=False)
    if "use_tc_tiling_on_sc" in pltpu.CompilerParams.__dataclass_fields__:
        cp = dataclasses.replace(cp, use_tc_tiling_on_sc=False)
    return pl.kernel(
        _sc_body,
        out_type=jax.ShapeDtypeStruct((P, C), jnp.float32),
        mesh=plsc.VectorSubcoreMesh(core_axis_name="c", subcore_axis_name="s"),
        compiler_params=cp,
        scratch_types=[
            pltpu.VMEM_SHARED((CHUNK + 1, C), jnp.float32),
            pltpu.VMEM((SCAN,), jnp.int32),
            pltpu.VMEM((SCAN,), jnp.float32),
            pltpu.VMEM((SCAN,), jnp.float32),
            pltpu.VMEM((SCAN,), jnp.int32),
            pltpu.VMEM((SCAN,), jnp.float32),
            pltpu.VMEM((SCAN,), jnp.float32),
            pltpu.VMEM((SCAN + TILE,), jnp.int32),
            pltpu.VMEM((SCAN + TILE,), jnp.int32),
            pltpu.VMEM((SCAN + TILE,), jnp.int32),
            pltpu.VMEM((SCAN + TILE,), jnp.float32),
            pltpu.VMEM((SCAN + TILE,), jnp.float32),
            pltpu.VMEM((TILE,), jnp.int32),
            pltpu.VMEM((TILE,), jnp.int32),
            pltpu.VMEM((TILE,), jnp.int32),
            pltpu.VMEM((TILE, C), jnp.float32),
            pltpu.VMEM((TILE, C), jnp.float32),
            pltpu.VMEM((TILE, C), jnp.float32),
            pltpu.VMEM((128, C), jnp.float32),
            pltpu.SemaphoreType.DMA,
            pltpu.SemaphoreType.DMA,
            pltpu.SemaphoreType.DMA,
            pltpu.SemaphoreType.DMA,
            pltpu.SemaphoreType.DMA,
            pltpu.SemaphoreType.DMA,
            pltpu.SemaphoreType.DMA,
        ],
    )


@jax.jit
def kernel(im0, flow):
    im0p = jnp.transpose(im0, (0, 2, 3, 1)).reshape(P, C)
    fx = flow[..., 0].reshape(B * H, W)
    fy = flow[..., 1].reshape(B * H, W)
    d2, wl2, wr2 = _build_records(fx, fy)
    outp = _get_sc_scatter()(im0p, d2.reshape(NREC), wl2.reshape(NREC),
                             wr2.reshape(NREC))
    return outp.reshape(B, H, W, C).transpose(0, 3, 1, 2)


# speedup vs baseline: 1.0857x; 1.0224x over previous
import dataclasses

import jax
import jax.numpy as jnp
from jax import lax
from jax.experimental import pallas as pl
from jax.experimental.pallas import tpu as pltpu
from jax.experimental.pallas import tpu_sc as plsc

B, C, H, W = 2, 96, 384, 384
P = B * H * W
NREC = 2 * P
NSUB = 16
NCORE = 2
REC_PER_SUB = NREC // NSUB
SCAN = 2048
NSCAN = REC_PER_SUB // SCAN
CHUNK = 32 * W
NPASS = (P // NCORE) // CHUNK
ROWS_PER_SUB = CHUNK // NSUB
TILE = 64


def _iota16():
    return lax.broadcasted_iota(jnp.int32, (16,), 0)


def _records_body(fx_ref, fy_ref, d_ref, wl_ref, wr_ref):
    i = pl.program_id(0)
    fx = fx_ref[...]
    fy = fy_ref[...]
    r_i = lax.broadcasted_iota(jnp.int32, (8, W), 0) + i * 8
    b = jnp.where(r_i >= H, 1, 0)
    y = r_i - b * H
    x = lax.broadcasted_iota(jnp.int32, (8, W), 1)
    fxc = jnp.clip(fx, -2.0 * W, 2.0 * W)
    fyc = jnp.clip(fy, -2.0 * W, 2.0 * W)
    gx = x.astype(jnp.float32) + fxc
    gy = y.astype(jnp.float32) + fyc
    x_f = jnp.floor(gx)
    y_f = jnp.floor(gy)
    x_c = x_f + 1.0
    y_c = y_f + 1.0
    nw = (x_c - gx) * (y_c - gy)
    ne = (gx - x_f) * (y_c - gy)
    sw = (x_c - gx) * (gy - y_f)
    se = (gx - x_f) * (gy - y_f)
    xfi = x_f.astype(jnp.int32)
    yfi = y_f.astype(jnp.int32)
    base = b * (H * W)
    vx = (xfi >= -1) & (xfi <= W - 1)
    dx = jnp.where(xfi >= 0, xfi, 0)
    for plane, (yp, wlr, wrr) in enumerate(((yfi, nw, ne), (yfi + 1, sw, se))):
        vy = (yp >= 0) & (yp < H)
        valid = vy & vx
        wl = jnp.where(xfi >= 0, wlr, wrr)
        wr = jnp.where((xfi >= 0) & (xfi <= W - 2), wrr, 0.0)
        d = jnp.where(valid, base + yp * W + dx, -1)
        d_ref[plane] = d
        wl_ref[plane] = jnp.where(valid, wl, 0.0)
        wr_ref[plane] = jnp.where(valid, wr, 0.0)


def _build_records(fx, fy):
    return pl.pallas_call(
        _records_body,
        grid=(B * H // 8,),
        in_specs=[pl.BlockSpec((8, W), lambda i: (i, 0)),
                  pl.BlockSpec((8, W), lambda i: (i, 0))],
        out_specs=[pl.BlockSpec((2, 8, W), lambda i: (0, i, 0)),
                   pl.BlockSpec((2, 8, W), lambda i: (0, i, 0)),
                   pl.BlockSpec((2, 8, W), lambda i: (0, i, 0))],
        out_shape=[jax.ShapeDtypeStruct((2, B * H, W), jnp.int32),
                   jax.ShapeDtypeStruct((2, B * H, W), jnp.float32),
                   jax.ShapeDtypeStruct((2, B * H, W), jnp.float32)],
    )(fx, fy)


def _sc_body(im0p, d2, wl2, wr2, outp,
             acc, scan_d0, scan_wl0, scan_wr0, scan_d1, scan_wl1, scan_wr1,
             ld, ld1, lp, lwl, lwr,
             ibp, ibd, ibd1, stag_in, stag_l, stag_r, zeros,
             zsem, da0, wa0, ra0, da1, wa1, ra1, ssl, ssr):
    c = lax.axis_index("c")
    s = lax.axis_index("s")
    p_base = (s % 8) * REC_PER_SUB
    rec_base = s * REC_PER_SUB

    z16f = jnp.zeros((16,), jnp.float32)
    z16i = jnp.zeros((16,), jnp.int32)

    @pl.loop(0, 128)
    def _zfill(r):
        for c6 in range(6):
            zeros[r, pl.ds(c6 * 16, 16)] = z16f

    scan_bufs = ((scan_d0, scan_wl0, scan_wr0, da0, wa0, ra0),
                 (scan_d1, scan_wl1, scan_wr1, da1, wa1, ra1))

    def _fetch(si, bufs):
        sd, swl, swr, dsm, wsm, rsm = bufs
        off = rec_base + jnp.minimum(si, NSCAN - 1) * SCAN
        pltpu.async_copy(d2.at[pl.ds(off, SCAN)], sd, dsm)
        pltpu.async_copy(wl2.at[pl.ds(off, SCAN)], swl, wsm)
        pltpu.async_copy(wr2.at[pl.ds(off, SCAN)], swr, rsm)

    def _wait(bufs):
        sd, swl, swr, dsm, wsm, rsm = bufs
        pltpu.make_async_copy(d2.at[pl.ds(0, SCAN)], sd, dsm).wait()
        pltpu.make_async_copy(wl2.at[pl.ds(0, SCAN)], swl, wsm).wait()
        pltpu.make_async_copy(wr2.at[pl.ds(0, SCAN)], swr, rsm).wait()

    @pl.loop(0, NPASS)
    def _pass(pi):
        chunk_lo = c * (P // NCORE) + pi * CHUNK

        zcs = []
        for k in range(ROWS_PER_SUB // 128):
            zcs.append(pltpu.async_copy(
                zeros, acc.at[pl.ds(s * ROWS_PER_SUB + k * 128, 128)], zsem))
        for zc in zcs:
            zc.wait()

        @pl.when(s == NSUB - 1)
        def _zero_guard():
            pltpu.sync_copy(zeros.at[pl.ds(0, 1)], acc.at[pl.ds(CHUNK, 1)])

        plsc.subcore_barrier()

        _fetch(jnp.int32(0), scan_bufs[0])
        for si in range(NSCAN):
            cur_bufs = scan_bufs[si % 2]
            _fetch(jnp.int32(si + 1), scan_bufs[(si + 1) % 2])
            _wait(cur_bufs)
            sd, swl, swr = cur_bufs[0], cur_bufs[1], cur_bufs[2]
            p0 = p_base + si * SCAN

            def vec_body(v, cursor, sd=sd, swl=swl, swr=swr, p0=p0):
                d = sd[pl.ds(v * 16, 16)]
                m = (d >= chunk_lo) & (d < chunk_lo + CHUNK)
                dl = d - chunk_lo
                plsc.store_compressed(ld.at[pl.ds(cursor, 16)], dl, mask=m)
                plsc.store_compressed(ld1.at[pl.ds(cursor, 16)], dl + 1,
                                      mask=m)
                pv = p0 + v * 16 + _iota16()
                plsc.store_compressed(lp.at[pl.ds(cursor, 16)], pv, mask=m)
                plsc.store_compressed(lwl.at[pl.ds(cursor, 16)],
                                      swl[pl.ds(v * 16, 16)], mask=m)
                plsc.store_compressed(lwr.at[pl.ds(cursor, 16)],
                                      swr[pl.ds(v * 16, 16)], mask=m)
                return cursor + jnp.sum(m.astype(jnp.int32))

            cursor = lax.fori_loop(0, SCAN // 16, vec_body, jnp.int32(0))

            @pl.loop(0, TILE // 16)
            def _pad(j):
                ld[pl.ds(cursor + j * 16, 16)] = z16i
                ld1[pl.ds(cursor + j * 16, 16)] = z16i
                lp[pl.ds(cursor + j * 16, 16)] = z16i
                lwl[pl.ds(cursor + j * 16, 16)] = z16f
                lwr[pl.ds(cursor + j * 16, 16)] = z16f

            ntiles = (cursor + TILE - 1) // TILE

            def tile_body(t, carry):
                tb = t * TILE

                @pl.loop(0, TILE // 16)
                def _cp(k):
                    ibp[pl.ds(k * 16, 16)] = lp[pl.ds(tb + k * 16, 16)]
                    ibd[pl.ds(k * 16, 16)] = ld[pl.ds(tb + k * 16, 16)]
                    ibd1[pl.ds(k * 16, 16)] = ld1[pl.ds(tb + k * 16, 16)]

                pltpu.sync_copy(im0p.at[ibp], stag_in)

                @pl.loop(0, TILE)
                def _scale(r):
                    idxv = jnp.full((16,), tb + r, jnp.int32)
                    wlv = plsc.load_gather(lwl, [idxv])
                    wrv = plsc.load_gather(lwr, [idxv])
                    for c6 in range(6):
                        v = stag_in[r, pl.ds(c6 * 16, 16)]
                        stag_l[r, pl.ds(c6 * 16, 16)] = v * wlv
                        stag_r[r, pl.ds(c6 * 16, 16)] = v * wrv

                sl = pltpu.async_copy(stag_l, acc.at[ibd], ssl, add=True)
                sr = pltpu.async_copy(stag_r, acc.at[ibd1], ssr, add=True)
                sl.wait()
                sr.wait()
                return carry

            lax.fori_loop(0, ntiles, tile_body, jnp.int32(0))

        _wait(scan_bufs[NSCAN % 2])

        plsc.subcore_barrier()
        pltpu.sync_copy(acc.at[pl.ds(s * ROWS_PER_SUB, ROWS_PER_SUB)],
                        outp.at[pl.ds(chunk_lo + s * ROWS_PER_SUB,
                                      ROWS_PER_SUB)])
        plsc.subcore_barrier()


_SC_SCATTER = None


def _get_sc_scatter():
    global _SC_SCATTER
    if _SC_SCATTER is None:
        _SC_SCATTER = _make_sc_scatter()
    return _SC_SCATTER


def _make_sc_scatter():
    cp = pltpu.CompilerParams()
    if "needs_layout_passes" in pltpu.CompilerParams.__dataclass_fields__:
        cp = dataclasses.replace(cp, needs_layout_passes=False)
    if "use_tc_tiling_on_sc" in pltpu.CompilerParams.__dataclass_fields__:
        cp = dataclasses.replace(cp, use_tc_tiling_on_sc=False)
    return pl.kernel(
        _sc_body,
        out_type=jax.ShapeDtypeStruct((P, C), jnp.float32),
        mesh=plsc.VectorSubcoreMesh(core_axis_name="c", subcore_axis_name="s"),
        compiler_params=cp,
        scratch_types=[
            pltpu.VMEM_SHARED((CHUNK + 1, C), jnp.float32),
            pltpu.VMEM((SCAN,), jnp.int32),
            pltpu.VMEM((SCAN,), jnp.float32),
            pltpu.VMEM((SCAN,), jnp.float32),
            pltpu.VMEM((SCAN,), jnp.int32),
            pltpu.VMEM((SCAN,), jnp.float32),
            pltpu.VMEM((SCAN,), jnp.float32),
            pltpu.VMEM((SCAN + TILE,), jnp.int32),
            pltpu.VMEM((SCAN + TILE,), jnp.int32),
            pltpu.VMEM((SCAN + TILE,), jnp.int32),
            pltpu.VMEM((SCAN + TILE,), jnp.float32),
            pltpu.VMEM((SCAN + TILE,), jnp.float32),
            pltpu.VMEM((TILE,), jnp.int32),
            pltpu.VMEM((TILE,), jnp.int32),
            pltpu.VMEM((TILE,), jnp.int32),
            pltpu.VMEM((TILE, C), jnp.float32),
            pltpu.VMEM((TILE, C), jnp.float32),
            pltpu.VMEM((TILE, C), jnp.float32),
            pltpu.VMEM((128, C), jnp.float32),
            pltpu.SemaphoreType.DMA,
            pltpu.SemaphoreType.DMA,
            pltpu.SemaphoreType.DMA,
            pltpu.SemaphoreType.DMA,
            pltpu.SemaphoreType.DMA,
            pltpu.SemaphoreType.DMA,
            pltpu.SemaphoreType.DMA,
            pltpu.SemaphoreType.DMA,
            pltpu.SemaphoreType.DMA,
        ],
    )


@jax.jit
def kernel(im0, flow):
    im0p = jnp.transpose(im0, (0, 2, 3, 1)).reshape(P, C)
    fx = flow[..., 0].reshape(B * H, W)
    fy = flow[..., 1].reshape(B * H, W)
    d2, wl2, wr2 = _build_records(fx, fy)
    outp = _get_sc_scatter()(im0p, d2.reshape(NREC), wl2.reshape(NREC),
                             wr2.reshape(NREC))
    return outp.reshape(B, H, W, C).transpose(0, 3, 1, 2)
